# baseline (speedup 1.0000x reference)
import math

import jax
import jax.numpy as jnp
import numpy as np
from jax import lax
from jax.experimental import pallas as pl
from jax.experimental.pallas import tpu as pltpu
from jax.experimental.pallas import tpu_sc as plsc

_NUM_VIEWS = 24
_NUM_INPUT = 16
_NUM_TARGET = 8

_NC = 2
_NS = 16
_NW_ACTIVE = 24


def _split_indices(total_views, num_input_views, num_target_views):
    g = math.gcd(num_input_views, num_target_views)
    group_size = total_views // g
    in_per_group = num_input_views // g
    tar_per_group = num_target_views // g
    input_indices = []
    target_indices = []
    for group_idx in range(g):
        start = group_idx * group_size
        block = list(range(start, start + group_size))
        input_indices.extend(block[:in_per_group])
        target_indices.extend(block[in_per_group:in_per_group + tar_per_group])
    input_indices = np.sort(np.array(input_indices, dtype=np.int32))
    target_indices = np.sort(np.array(target_indices, dtype=np.int32))
    return input_indices, target_indices


def _tc_copy(src_ref, dst_ref):
    dst_ref[...] = src_ref[...]


def _sc_target_copy(n_chunks, src_hbm, tar_hbm, buf0, buf1, ls0, ls1, ss0, ss1):
    c = lax.axis_index("c")
    s = lax.axis_index("s")
    wid = s * _NC + c
    base = wid * n_chunks
    bufs = (buf0, buf1)
    lsems = (ls0, ls1)
    ssems = (ss0, ss1)
    active = wid < _NW_ACTIVE

    def mk(i):
        k = base + i
        src_row = 18 * (k // 6) + 12 + k % 6
        ld = pltpu.make_async_copy(
            src_hbm.at[pl.ds(src_row, 1)], bufs[i % 2], lsems[i % 2])
        st = pltpu.make_async_copy(
            bufs[i % 2], tar_hbm.at[pl.ds(k, 1)], ssems[i % 2])
        return ld, st

    @pl.when(active)
    def _():
        ops = [mk(i) for i in range(n_chunks)]
        ops[0][0].start()
        for i in range(n_chunks):
            ld, st = ops[i]
            ld.wait()
            st.start()
            if i + 1 < n_chunks:
                if i >= 1:
                    ops[i - 1][1].wait()
                ops[i + 1][0].start()
        if n_chunks >= 2:
            ops[n_chunks - 2][1].wait()
        ops[n_chunks - 1][1].wait()


def kernel(image):
    B, V, C, H, W = image.shape
    G = V // 3
    rows = B * G
    CH = C * H
    groups = image.reshape(rows, 3 * CH, W)

    gpb = 4
    input_flat = pl.pallas_call(
        _tc_copy,
        grid=(rows // gpb,),
        in_specs=[pl.BlockSpec((gpb, 2 * CH, W), lambda r: (r, 0, 0))],
        out_specs=pl.BlockSpec((gpb, 2 * CH, W), lambda r: (r, 0, 0)),
        out_shape=jax.ShapeDtypeStruct((rows, 2 * CH, W), image.dtype),
    )(groups)

    chunks = image.reshape(B * V * C * 2, H // 2, W)
    n_tar_chunks = B * G * C * 2
    n_chunks = n_tar_chunks // _NW_ACTIVE
    mesh = plsc.VectorSubcoreMesh(core_axis_name="c", subcore_axis_name="s")
    target_flat = pl.kernel(
        lambda *refs: _sc_target_copy(n_chunks, *refs),
        out_type=jax.ShapeDtypeStruct((n_tar_chunks, H // 2, W), image.dtype),
        mesh=mesh,
        compiler_params=pltpu.CompilerParams(
            use_tc_tiling_on_sc=True, skip_device_barrier=True),
        scratch_types=[
            pltpu.VMEM((1, H // 2, W), jnp.float32),
            pltpu.VMEM((1, H // 2, W), jnp.float32),
            pltpu.SemaphoreType.DMA,
            pltpu.SemaphoreType.DMA,
            pltpu.SemaphoreType.DMA,
            pltpu.SemaphoreType.DMA,
        ],
    )(chunks)

    input_image = input_flat.reshape(B, 2 * G, C, H, W)
    target_image = target_flat.reshape(B, G, C, H, W)

    ii, ti = _split_indices(_NUM_VIEWS, _NUM_INPUT, _NUM_TARGET)
    input_pattern = jnp.tile(jnp.asarray(ii)[None, :], (B, 1))
    target_pattern = jnp.tile(jnp.asarray(ti)[None, :], (B, 1))
    return (input_image, target_image, input_pattern, target_pattern)


# speedup vs baseline: 1.0122x; 1.0122x over previous
import math

import jax
import jax.numpy as jnp
import numpy as np
from jax import lax
from jax.experimental import pallas as pl
from jax.experimental.pallas import tpu as pltpu
from jax.experimental.pallas import tpu_sc as plsc

_NUM_VIEWS = 24
_NUM_INPUT = 16
_NUM_TARGET = 8

_NC = 2
_NS = 16
_NW_ACTIVE = 24


def _split_indices(total_views, num_input_views, num_target_views):
    g = math.gcd(num_input_views, num_target_views)
    group_size = total_views // g
    in_per_group = num_input_views // g
    tar_per_group = num_target_views // g
    input_indices = []
    target_indices = []
    for group_idx in range(g):
        start = group_idx * group_size
        block = list(range(start, start + group_size))
        input_indices.extend(block[:in_per_group])
        target_indices.extend(block[in_per_group:in_per_group + tar_per_group])
    input_indices = np.sort(np.array(input_indices, dtype=np.int32))
    target_indices = np.sort(np.array(target_indices, dtype=np.int32))
    return input_indices, target_indices


def _tc_copy(src_ref, dst_ref):
    dst_ref[...] = src_ref[...]


def _sc_target_copy(n_chunks, src_hbm, tar_hbm, buf0, buf1, ls0, ls1, ss0, ss1):
    c = lax.axis_index("c")
    s = lax.axis_index("s")
    wid = s * _NC + c
    base = wid * n_chunks
    bufs = (buf0, buf1)
    lsems = (ls0, ls1)
    ssems = (ss0, ss1)
    active = wid < _NW_ACTIVE

    def mk(i):
        k = base + i
        src_row = 18 * (k // 6) + 12 + k % 6
        ld = pltpu.make_async_copy(
            src_hbm.at[pl.ds(src_row, 1)], bufs[i % 2], lsems[i % 2])
        st = pltpu.make_async_copy(
            bufs[i % 2], tar_hbm.at[pl.ds(k, 1)], ssems[i % 2])
        return ld, st

    @pl.when(active)
    def _():
        ops = [mk(i) for i in range(n_chunks)]
        ops[0][0].start()
        for i in range(n_chunks):
            ld, st = ops[i]
            ld.wait()
            st.start()
            if i + 1 < n_chunks:
                if i >= 1:
                    ops[i - 1][1].wait()
                ops[i + 1][0].start()
        if n_chunks >= 2:
            ops[n_chunks - 2][1].wait()
        ops[n_chunks - 1][1].wait()


def kernel(image):
    B, V, C, H, W = image.shape
    G = V // 3
    rows = B * G
    CH = C * H
    groups = image.reshape(rows, 3 * CH, W)

    gpb = 8
    input_flat = pl.pallas_call(
        _tc_copy,
        grid=(rows // gpb,),
        in_specs=[pl.BlockSpec((gpb, 2 * CH, W), lambda r: (r, 0, 0))],
        out_specs=pl.BlockSpec((gpb, 2 * CH, W), lambda r: (r, 0, 0)),
        out_shape=jax.ShapeDtypeStruct((rows, 2 * CH, W), image.dtype),
        compiler_params=pltpu.CompilerParams(vmem_limit_bytes=56623104),
    )(groups)

    chunks = image.reshape(B * V * C * 2, H // 2, W)
    n_tar_chunks = B * G * C * 2
    n_chunks = n_tar_chunks // _NW_ACTIVE
    mesh = plsc.VectorSubcoreMesh(core_axis_name="c", subcore_axis_name="s")
    target_flat = pl.kernel(
        lambda *refs: _sc_target_copy(n_chunks, *refs),
        out_type=jax.ShapeDtypeStruct((n_tar_chunks, H // 2, W), image.dtype),
        mesh=mesh,
        compiler_params=pltpu.CompilerParams(use_tc_tiling_on_sc=True),
        scratch_types=[
            pltpu.VMEM((1, H // 2, W), jnp.float32),
            pltpu.VMEM((1, H // 2, W), jnp.float32),
            pltpu.SemaphoreType.DMA,
            pltpu.SemaphoreType.DMA,
            pltpu.SemaphoreType.DMA,
            pltpu.SemaphoreType.DMA,
        ],
    )(chunks)

    input_image = input_flat.reshape(B, 2 * G, C, H, W)
    target_image = target_flat.reshape(B, G, C, H, W)

    ii, ti = _split_indices(_NUM_VIEWS, _NUM_INPUT, _NUM_TARGET)
    input_pattern = jnp.tile(jnp.asarray(ii)[None, :], (B, 1))
    target_pattern = jnp.tile(jnp.asarray(ti)[None, :], (B, 1))
    return (input_image, target_image, input_pattern, target_pattern)
